# Initial kernel scaffold; baseline (speedup 1.0000x reference)
import functools

import jax
import jax.numpy as jnp
from jax import lax
from jax.experimental import pallas as pl
from jax.experimental.pallas import tpu as pltpu
from jax.experimental.pallas import tpu_sc as plsc

N = 10000
E = 320000
D_IN = 128
D_H = 128
D_OUT = 64
NEG = 0.2

NC = 2
NS = 16
NT = NC * NS
EPT = E // NT
CH = 80
NCH = EPT // CH
RPS = N // NS
LPS = N // 10

_mesh = plsc.VectorSubcoreMesh(core_axis_name="c", subcore_axis_name="s")


def _leaky_exp(x):
    return jnp.exp(jnp.where(x >= 0, x, x * NEG))


def _sc_stage1(feat, el, er, src3, dst3, zeros2, zeros1):
    kern = pl.kernel(
        _sc_stage1_body,
        out_type=(
            jax.ShapeDtypeStruct((NT, NCH, CH), jnp.float32),
            jax.ShapeDtypeStruct((NC, N), jnp.float32),
            jax.ShapeDtypeStruct((NC, N, D_H), jnp.float32),
        ),
        mesh=_mesh,
        scratch_types=[
            pltpu.VMEM((N,), jnp.float32),
            pltpu.VMEM((N,), jnp.float32),
            pltpu.VMEM((NCH, CH), jnp.int32),
            pltpu.VMEM((NCH, CH), jnp.int32),
            pltpu.VMEM((NCH, CH), jnp.float32),
            pltpu.VMEM((CH, D_H), jnp.float32),
            pltpu.VMEM_SHARED((N,), jnp.float32),
            pltpu.VMEM_SHARED((N, D_H), jnp.float32),
            pltpu.SemaphoreType.DMA,
        ],
    )
    return kern(feat, el, er, src3, dst3, zeros2, zeros1)


def _sc_stage1_body(feat_hbm, el_hbm, er_hbm, src_hbm, dst_hbm, z2_hbm, z1_hbm,
                    ex_hbm, s_hbm, p_hbm,
                    el_v, er_v, src_v, dst_v, ex_v, rows_v, s_sh, acc_sh, sem):
    core = lax.axis_index("c")
    sub = lax.axis_index("s")
    tile = core * NS + sub

    pltpu.sync_copy(el_hbm, el_v)
    pltpu.sync_copy(er_hbm, er_v)
    pltpu.sync_copy(src_hbm.at[tile], src_v)
    pltpu.sync_copy(dst_hbm.at[tile], dst_v)

    pltpu.sync_copy(z2_hbm.at[pl.ds(sub * RPS, RPS)], acc_sh.at[pl.ds(sub * RPS, RPS)])

    @pl.when(sub < 10)
    def _():
        pltpu.sync_copy(z1_hbm.at[pl.ds(sub * LPS, LPS)], s_sh.at[pl.ds(sub * LPS, LPS)])

    plsc.subcore_barrier()

    @pl.loop(0, NCH)
    def _(j):
        @pl.loop(0, CH, step=16)
        def _(i):
            s16 = src_v[j, pl.ds(i, 16)]
            d16 = dst_v[j, pl.ds(i, 16)]
            vals = plsc.load_gather(el_v, [s16]) + plsc.load_gather(er_v, [d16])
            ex_v[j, pl.ds(i, 16)] = _leaky_exp(vals)

        pltpu.sync_copy(ex_v.at[j], s_sh.at[dst_v.at[j]], add=True)

    pltpu.sync_copy(ex_v, ex_hbm.at[tile])

    @pl.loop(0, NCH)
    def _(j):
        pltpu.sync_copy(feat_hbm.at[src_v.at[j]], rows_v)

        @pl.loop(0, CH)
        def _(i):
            exb = plsc.load_gather(
                ex_v, [jnp.full((16,), j, jnp.int32), jnp.full((16,), i, jnp.int32)]
            )
            for dd in range(D_H // 16):
                rows_v[i, pl.ds(dd * 16, 16)] = rows_v[i, pl.ds(dd * 16, 16)] * exb

        pltpu.sync_copy(rows_v, acc_sh.at[dst_v.at[j]], add=True)

    plsc.subcore_barrier()

    pltpu.sync_copy(acc_sh.at[pl.ds(sub * RPS, RPS)],
                    p_hbm.at[core, pl.ds(sub * RPS, RPS)])

    @pl.when(sub < 10)
    def _():
        pltpu.sync_copy(s_sh.at[pl.ds(sub * LPS, LPS)],
                        s_hbm.at[core, pl.ds(sub * LPS, LPS)])


def _sc_prop(x, src3, dst3, ex3, zeros2, d):
    kern = pl.kernel(
        functools.partial(_sc_prop_body, d=d),
        out_type=jax.ShapeDtypeStruct((NC, N, d), jnp.float32),
        mesh=_mesh,
        scratch_types=[
            pltpu.VMEM((NCH, CH), jnp.int32),
            pltpu.VMEM((NCH, CH), jnp.int32),
            pltpu.VMEM((NCH, CH), jnp.float32),
            pltpu.VMEM((CH, d), jnp.float32),
            pltpu.VMEM_SHARED((N, d), jnp.float32),
            pltpu.SemaphoreType.DMA,
        ],
    )
    return kern(x, src3, dst3, ex3, zeros2)


def _sc_prop_body(x_hbm, src_hbm, dst_hbm, ex_hbm, z2_hbm, out_hbm,
                  src_v, dst_v, ex_v, rows_v, acc_sh, sem, *, d):
    core = lax.axis_index("c")
    sub = lax.axis_index("s")
    tile = core * NS + sub

    pltpu.sync_copy(src_hbm.at[tile], src_v)
    pltpu.sync_copy(dst_hbm.at[tile], dst_v)
    pltpu.sync_copy(ex_hbm.at[tile], ex_v)
    pltpu.sync_copy(z2_hbm.at[pl.ds(sub * RPS, RPS)], acc_sh.at[pl.ds(sub * RPS, RPS)])
    plsc.subcore_barrier()

    @pl.loop(0, NCH)
    def _(j):
        pltpu.sync_copy(x_hbm.at[src_v.at[j]], rows_v)

        @pl.loop(0, CH)
        def _(i):
            exb = plsc.load_gather(
                ex_v, [jnp.full((16,), j, jnp.int32), jnp.full((16,), i, jnp.int32)]
            )
            for dd in range(d // 16):
                rows_v[i, pl.ds(dd * 16, 16)] = rows_v[i, pl.ds(dd * 16, 16)] * exb

        pltpu.sync_copy(rows_v, acc_sh.at[dst_v.at[j]], add=True)

    plsc.subcore_barrier()
    pltpu.sync_copy(acc_sh.at[pl.ds(sub * RPS, RPS)],
                    out_hbm.at[core, pl.ds(sub * RPS, RPS)])


def _dot(a, b):
    return jnp.dot(a, b, preferred_element_type=jnp.float32,
                   precision=lax.Precision.HIGHEST)


def _tc_feat_body(x_ref, wg_ref, al_ref, ar_ref, feat_ref, el_ref, er_ref):
    f = _dot(x_ref[...], wg_ref[...])
    feat_ref[...] = f
    el_ref[...] = _dot(f, al_ref[...])
    er_ref[...] = _dot(f, ar_ref[...])


def _tc_feat(x, wg, al, ar):
    return pl.pallas_call(
        _tc_feat_body,
        out_shape=(
            jax.ShapeDtypeStruct((N, D_H), jnp.float32),
            jax.ShapeDtypeStruct((N, 1), jnp.float32),
            jax.ShapeDtypeStruct((N, 1), jnp.float32),
        ),
    )(x, wg, al, ar)


def _tc_h1_body(p_ref, s_ref, b_ref, o_ref):
    inv = 1.0 / (s_ref[0] + s_ref[1] + 1e-9)
    t = (p_ref[0] + p_ref[1]) * inv + b_ref[...]
    o_ref[...] = jnp.maximum(t, 0.0)


def _tc_h1(p, s, b_gat):
    return pl.pallas_call(
        _tc_h1_body,
        out_shape=jax.ShapeDtypeStruct((N, D_H), jnp.float32),
    )(p, s, b_gat)


def _tc_h2p_body(q_ref, s_ref, w1_ref, b1_ref, w2_ref, o_ref):
    inv = 1.0 / (s_ref[0] + s_ref[1] + 1e-9)
    t2 = (q_ref[0] + q_ref[1]) * inv
    h2 = _dot(t2, w1_ref[...]) + b1_ref[...]
    o_ref[...] = _dot(h2, w2_ref[...])


def _tc_h2p(q, s, w1, b1, w2):
    return pl.pallas_call(
        _tc_h2p_body,
        out_shape=jax.ShapeDtypeStruct((N, D_OUT), jnp.float32),
    )(q, s, w1, b1, w2)


def _tc_out_body(r_ref, s_ref, b2_ref, o_ref):
    inv = 1.0 / (s_ref[0] + s_ref[1] + 1e-9)
    o_ref[...] = (r_ref[0] + r_ref[1]) * inv + b2_ref[...]


def _tc_out(r, s, b2):
    return pl.pallas_call(
        _tc_out_body,
        out_shape=jax.ShapeDtypeStruct((N, D_OUT), jnp.float32),
    )(r, s, b2)


def kernel(inputs, edge_index, W_gat, attn_l, attn_r, b_gat, W1, b1, W2, b2):
    src3 = edge_index[0].reshape(NT, NCH, CH)
    dst3 = edge_index[1].reshape(NT, NCH, CH)
    zeros2 = jnp.zeros((N, D_H), jnp.float32)
    zeros1 = jnp.zeros((N,), jnp.float32)

    feat, el2, er2 = _tc_feat(inputs, W_gat,
                              attn_l.reshape(D_H, 1), attn_r.reshape(D_H, 1))
    el = el2.reshape(N)
    er = er2.reshape(N)

    ex3, s, p = _sc_stage1(feat, el, er, src3, dst3, zeros2, zeros1)
    s3 = s.reshape(NC, N, 1)

    h1 = _tc_h1(p, s3, b_gat.reshape(1, D_H))

    q = _sc_prop(h1, src3, dst3, ex3, zeros2, D_H)
    h2p = _tc_h2p(q, s3, W1, b1.reshape(1, D_H), W2)

    r = _sc_prop(h2p, src3, dst3, ex3, zeros2[:, :D_OUT], D_OUT)
    logits = _tc_out(r, s3, b2.reshape(1, D_OUT))
    return logits


# speedup vs baseline: 7.0846x; 7.0846x over previous
import dataclasses
import functools

import jax
import jax.numpy as jnp
from jax import lax
from jax.experimental import pallas as pl
from jax.experimental.pallas import tpu as pltpu
from jax.experimental.pallas import tpu_sc as plsc

N = 10000
E = 320000
D_IN = 128
D_H = 128
D_OUT = 64
NEG = 0.2

NC = 2
NS = 16
NT = NC * NS
CH = 128
NCH = 80
EPTP = NCH * CH
EPAD = NT * EPTP
NP = 10240
RPS = NP // NS

_mesh = plsc.VectorSubcoreMesh(core_axis_name="c", subcore_axis_name="s")

_cp = pltpu.CompilerParams()
if "needs_layout_passes" in pltpu.CompilerParams.__dataclass_fields__:
    _cp = dataclasses.replace(_cp, needs_layout_passes=False)


def _leaky_exp(x):
    return jnp.exp(jnp.where(x >= 0, x, x * NEG))


def _sc_attn(el, er, src3, dst3, zeros1):
    kern = pl.kernel(
        _sc_attn_body,
        out_type=(
            jax.ShapeDtypeStruct((NT, NCH, CH), jnp.float32),
            jax.ShapeDtypeStruct((NC, NP), jnp.float32),
        ),
        mesh=_mesh,
        scratch_types=[
            pltpu.VMEM((N,), jnp.float32),
            pltpu.VMEM((N,), jnp.float32),
            pltpu.VMEM((NCH, CH), jnp.int32),
            pltpu.VMEM((NCH, CH), jnp.int32),
            pltpu.VMEM((NCH, CH), jnp.float32),
            pltpu.VMEM_SHARED((NP,), jnp.float32),
            pltpu.SemaphoreType.DMA,
        ],
        compiler_params=_cp,
    )
    return kern(el, er, src3, dst3, zeros1)


def _sc_attn_body(el_hbm, er_hbm, src_hbm, dst_hbm, z1_hbm,
                  ex_hbm, s_hbm,
                  el_v, er_v, src_v, dst_v, ex_v, s_sh, sem):
    core = lax.axis_index("c")
    sub = lax.axis_index("s")
    tile = core * NS + sub

    pltpu.sync_copy(el_hbm, el_v)
    pltpu.sync_copy(er_hbm, er_v)
    pltpu.sync_copy(src_hbm.at[tile], src_v)
    pltpu.sync_copy(dst_hbm.at[tile], dst_v)

    pltpu.sync_copy(z1_hbm.at[pl.ds(sub * RPS, RPS)], s_sh.at[pl.ds(sub * RPS, RPS)])
    plsc.subcore_barrier()

    lane = lax.iota(jnp.int32, 16)

    @pl.loop(0, NCH)
    def _(j):
        @pl.loop(0, CH, step=16)
        def _(i):
            s16 = src_v[j, pl.ds(i, 16)]
            d16 = dst_v[j, pl.ds(i, 16)]
            vals = plsc.load_gather(el_v, [s16]) + plsc.load_gather(er_v, [d16])
            gid = tile * EPTP + j * CH + i + lane
            ex_v[j, pl.ds(i, 16)] = jnp.where(gid < E, _leaky_exp(vals), 0.0)

        pltpu.sync_copy(ex_v.at[j], s_sh.at[dst_v.at[j]], add=True)

    pltpu.sync_copy(ex_v, ex_hbm.at[tile])

    plsc.subcore_barrier()
    pltpu.sync_copy(s_sh.at[pl.ds(sub * RPS, RPS)],
                    s_hbm.at[core, pl.ds(sub * RPS, RPS)])


def _sc_prop(x, src3, dst3, ex3, zeros2, d):
    kern = pl.kernel(
        functools.partial(_sc_prop_body, d=d),
        out_type=jax.ShapeDtypeStruct((NC, NP, d), jnp.float32),
        mesh=_mesh,
        scratch_types=[
            pltpu.VMEM((NCH, CH), jnp.int32),
            pltpu.VMEM((NCH, CH), jnp.int32),
            pltpu.VMEM((NCH, CH), jnp.float32),
            pltpu.VMEM((CH, d), jnp.float32),
            pltpu.VMEM_SHARED((NP, d), jnp.float32),
            pltpu.SemaphoreType.DMA,
        ],
        compiler_params=_cp,
    )
    return kern(x, src3, dst3, ex3, zeros2)


def _sc_prop_body(x_hbm, src_hbm, dst_hbm, ex_hbm, z2_hbm, out_hbm,
                  src_v, dst_v, ex_v, rows_v, acc_sh, sem, *, d):
    core = lax.axis_index("c")
    sub = lax.axis_index("s")
    tile = core * NS + sub

    pltpu.sync_copy(src_hbm.at[tile], src_v)
    pltpu.sync_copy(dst_hbm.at[tile], dst_v)
    pltpu.sync_copy(ex_hbm.at[tile], ex_v)
    pltpu.sync_copy(z2_hbm.at[pl.ds(sub * RPS, RPS)],
                    acc_sh.at[pl.ds(sub * RPS, RPS)])
    plsc.subcore_barrier()

    @pl.loop(0, NCH)
    def _(j):
        pltpu.sync_copy(x_hbm.at[src_v.at[j]], rows_v)

        @pl.loop(0, CH)
        def _(i):
            exb = plsc.load_gather(
                ex_v, [jnp.full((16,), j, jnp.int32), jnp.full((16,), i, jnp.int32)]
            )
            for dd in range(d // 16):
                rows_v[i, pl.ds(dd * 16, 16)] = rows_v[i, pl.ds(dd * 16, 16)] * exb

        pltpu.sync_copy(rows_v, acc_sh.at[dst_v.at[j]], add=True)

    plsc.subcore_barrier()
    pltpu.sync_copy(acc_sh.at[pl.ds(sub * RPS, RPS)],
                    out_hbm.at[core, pl.ds(sub * RPS, RPS)])


def _dot(a, b):
    return jnp.dot(a, b, preferred_element_type=jnp.float32,
                   precision=lax.Precision.HIGHEST)


def _tc_feat_body(x_ref, wg_ref, al_ref, ar_ref, feat_ref, el_ref, er_ref):
    f = _dot(x_ref[...], wg_ref[...])
    feat_ref[...] = f
    el_ref[...] = _dot(f, al_ref[...])
    er_ref[...] = _dot(f, ar_ref[...])


def _tc_feat(x, wg, al, ar):
    return pl.pallas_call(
        _tc_feat_body,
        out_shape=(
            jax.ShapeDtypeStruct((N, D_H), jnp.float32),
            jax.ShapeDtypeStruct((N, 1), jnp.float32),
            jax.ShapeDtypeStruct((N, 1), jnp.float32),
        ),
    )(x, wg, al, ar)


def _tc_h1_body(p_ref, s_ref, b_ref, o_ref):
    inv = 1.0 / (s_ref[0, :N] + s_ref[1, :N] + 1e-9)
    t = (p_ref[0, :N] + p_ref[1, :N]) * inv + b_ref[...]
    o_ref[...] = jnp.maximum(t, 0.0)


def _tc_h1(p, s, b_gat):
    return pl.pallas_call(
        _tc_h1_body,
        out_shape=jax.ShapeDtypeStruct((N, D_H), jnp.float32),
    )(p, s, b_gat)


def _tc_h2_body(q_ref, s_ref, w1_ref, b1_ref, o_ref):
    inv = 1.0 / (s_ref[0, :N] + s_ref[1, :N] + 1e-9)
    t2 = (q_ref[0, :N] + q_ref[1, :N]) * inv
    o_ref[...] = _dot(t2, w1_ref[...]) + b1_ref[...]


def _tc_h2(q, s, w1, b1):
    return pl.pallas_call(
        _tc_h2_body,
        out_shape=jax.ShapeDtypeStruct((N, D_H), jnp.float32),
    )(q, s, w1, b1)


def _tc_out_body(r_ref, s_ref, w2_ref, b2_ref, o_ref):
    inv = 1.0 / (s_ref[0, :N] + s_ref[1, :N] + 1e-9)
    o_ref[...] = _dot((r_ref[0, :N] + r_ref[1, :N]) * inv, w2_ref[...]) + b2_ref[...]


def _tc_out(r, s, w2, b2):
    return pl.pallas_call(
        _tc_out_body,
        out_shape=jax.ShapeDtypeStruct((N, D_OUT), jnp.float32),
    )(r, s, w2, b2)


def kernel(inputs, edge_index, W_gat, attn_l, attn_r, b_gat, W1, b1, W2, b2):
    pad = EPAD - E
    src3 = jnp.pad(edge_index[0], (0, pad)).reshape(NT, NCH, CH)
    dst3 = jnp.pad(edge_index[1], (0, pad)).reshape(NT, NCH, CH)
    zeros2 = jnp.zeros((NP, D_H), jnp.float32)
    zeros1 = jnp.zeros((NP,), jnp.float32)

    feat, el2, er2 = _tc_feat(inputs, W_gat,
                              attn_l.reshape(D_H, 1), attn_r.reshape(D_H, 1))
    el = el2.reshape(N)
    er = er2.reshape(N)

    ex3, s = _sc_attn(el, er, src3, dst3, zeros1)
    s3 = s.reshape(NC, NP, 1)

    p = _sc_prop(feat, src3, dst3, ex3, zeros2, D_H)
    h1 = _tc_h1(p, s3, b_gat.reshape(1, D_H))

    q = _sc_prop(h1, src3, dst3, ex3, zeros2, D_H)
    h2 = _tc_h2(q, s3, W1, b1.reshape(1, D_H))

    r = _sc_prop(h2, src3, dst3, ex3, zeros2, D_H)
    logits = _tc_out(r, s3, W2, b2.reshape(1, D_OUT))
    return logits


# speedup vs baseline: 8.4316x; 1.1901x over previous
import dataclasses
import functools

import jax
import jax.numpy as jnp
from jax import lax
from jax.experimental import pallas as pl
from jax.experimental.pallas import tpu as pltpu
from jax.experimental.pallas import tpu_sc as plsc

N = 10000
E = 320000
D_IN = 128
D_H = 128
D_OUT = 64
NEG = 0.2

NC = 2
NS = 16
NT = NC * NS
CH = 128
CH2 = 64
NCH = 80
EPTP = NCH * CH
EPAD = NT * EPTP
NP = 10240
RPS = NP // NS

_mesh = plsc.VectorSubcoreMesh(core_axis_name="c", subcore_axis_name="s")

_cp = pltpu.CompilerParams()
if "needs_layout_passes" in pltpu.CompilerParams.__dataclass_fields__:
    _cp = dataclasses.replace(_cp, needs_layout_passes=False)


def _leaky_exp(x):
    return jnp.exp(jnp.where(x >= 0, x, x * NEG))


def _sc_attn(el, er, src3, dst3, zeros1):
    kern = pl.kernel(
        _sc_attn_body,
        out_type=(
            jax.ShapeDtypeStruct((NT, NCH, CH), jnp.float32),
            jax.ShapeDtypeStruct((NC, NP), jnp.float32),
        ),
        mesh=_mesh,
        scratch_types=[
            pltpu.VMEM((N,), jnp.float32),
            pltpu.VMEM((N,), jnp.float32),
            pltpu.VMEM((NCH, CH), jnp.int32),
            pltpu.VMEM((NCH, CH), jnp.int32),
            pltpu.VMEM((NCH, CH), jnp.float32),
            pltpu.VMEM_SHARED((NP,), jnp.float32),
            pltpu.SemaphoreType.DMA,
        ],
        compiler_params=_cp,
    )
    return kern(el, er, src3, dst3, zeros1)


def _sc_attn_body(el_hbm, er_hbm, src_hbm, dst_hbm, z1_hbm,
                  ex_hbm, s_hbm,
                  el_v, er_v, src_v, dst_v, ex_v, s_sh, sem):
    core = lax.axis_index("c")
    sub = lax.axis_index("s")
    tile = core * NS + sub

    pltpu.sync_copy(el_hbm, el_v)
    pltpu.sync_copy(er_hbm, er_v)
    pltpu.sync_copy(src_hbm.at[tile], src_v)
    pltpu.sync_copy(dst_hbm.at[tile], dst_v)

    pltpu.sync_copy(z1_hbm.at[pl.ds(sub * RPS, RPS)], s_sh.at[pl.ds(sub * RPS, RPS)])
    plsc.subcore_barrier()

    lane = lax.iota(jnp.int32, 16)

    @pl.loop(0, NCH)
    def _(j):
        @pl.loop(0, CH, step=16)
        def _(i):
            s16 = src_v[j, pl.ds(i, 16)]
            d16 = dst_v[j, pl.ds(i, 16)]
            vals = plsc.load_gather(el_v, [s16]) + plsc.load_gather(er_v, [d16])
            gid = tile * EPTP + j * CH + i + lane
            ex_v[j, pl.ds(i, 16)] = jnp.where(gid < E, _leaky_exp(vals), 0.0)

        pltpu.sync_copy(ex_v.at[j], s_sh.at[dst_v.at[j]], add=True)

    pltpu.sync_copy(ex_v, ex_hbm.at[tile])

    plsc.subcore_barrier()
    pltpu.sync_copy(s_sh.at[pl.ds(sub * RPS, RPS)],
                    s_hbm.at[core, pl.ds(sub * RPS, RPS)])


def _sc_prop(x, src3, dst3, ex3, zeros2, d):
    kern = pl.kernel(
        functools.partial(_sc_prop_body, d=d),
        out_type=jax.ShapeDtypeStruct((NC, NP, d), jnp.float32),
        mesh=_mesh,
        scratch_types=[
            pltpu.VMEM((NCH, CH), jnp.int32),
            pltpu.VMEM((NCH, CH), jnp.int32),
            pltpu.VMEM((NCH, CH), jnp.float32),
            pltpu.VMEM((CH2, d), jnp.float32),
            pltpu.VMEM((CH2, d), jnp.float32),
            pltpu.VMEM((2, CH2), jnp.int32),
            pltpu.VMEM_SHARED((NP, d), jnp.float32),
            pltpu.SemaphoreType.DMA,
            pltpu.SemaphoreType.DMA,
            pltpu.SemaphoreType.DMA,
            pltpu.SemaphoreType.DMA,
        ],
        compiler_params=_cp,
    )
    return kern(x, src3, dst3, ex3, zeros2)


def _sc_prop_body(x_hbm, src_hbm, dst_hbm, ex_hbm, z2_hbm, out_hbm,
                  src_v, dst_v, ex_v, rows0_v, rows1_v, dstw_v, acc_sh,
                  gsem0, gsem1, ssem0, ssem1, *, d):
    core = lax.axis_index("c")
    sub = lax.axis_index("s")
    tile = core * NS + sub

    pltpu.sync_copy(src_hbm.at[tile], src_v)
    pltpu.sync_copy(dst_hbm.at[tile], dst_v)
    pltpu.sync_copy(ex_hbm.at[tile], ex_v)
    pltpu.sync_copy(z2_hbm.at[pl.ds(sub * RPS, RPS)],
                    acc_sh.at[pl.ds(sub * RPS, RPS)])
    plsc.subcore_barrier()

    def g_start(j, h, rows, sem):
        pltpu.async_copy(x_hbm.at[src_v.at[j, pl.ds(h * CH2, CH2)]], rows, sem)

    def g_wait(rows, sem):
        pltpu.make_async_copy(x_hbm.at[src_v.at[0, pl.ds(0, CH2)]], rows, sem).wait()

    def s_start(j, h, p, rows, sem):
        @pl.loop(0, CH2, step=16)
        def _(t):
            dstw_v[p, pl.ds(t, 16)] = dst_v[j, pl.ds(h * CH2 + t, 16)]

        pltpu.async_copy(rows, acc_sh.at[dstw_v.at[p]], sem, add=True)

    def s_wait(rows, sem):
        pltpu.make_async_copy(rows, acc_sh.at[dstw_v.at[0]], sem).wait()

    def scale(j, h, rows):
        @pl.loop(0, CH2)
        def _(i):
            exb = plsc.load_gather(
                ex_v,
                [jnp.full((16,), j, jnp.int32),
                 jnp.full((16,), h * CH2 + i, jnp.int32)],
            )
            for dd in range(d // 16):
                rows[i, pl.ds(dd * 16, 16)] = rows[i, pl.ds(dd * 16, 16)] * exb

    g_start(0, 0, rows0_v, gsem0)

    @pl.loop(0, NCH)
    def _(j):
        g_wait(rows0_v, gsem0)

        @pl.when(j > 0)
        def _():
            s_wait(rows1_v, ssem1)

        g_start(j, 1, rows1_v, gsem1)
        scale(j, 0, rows0_v)
        s_start(j, 0, 0, rows0_v, ssem0)

        g_wait(rows1_v, gsem1)
        s_wait(rows0_v, ssem0)

        @pl.when(j + 1 < NCH)
        def _():
            g_start(j + 1, 0, rows0_v, gsem0)

        scale(j, 1, rows1_v)
        s_start(j, 1, 1, rows1_v, ssem1)

    s_wait(rows1_v, ssem1)

    plsc.subcore_barrier()
    pltpu.sync_copy(acc_sh.at[pl.ds(sub * RPS, RPS)],
                    out_hbm.at[core, pl.ds(sub * RPS, RPS)])


def _dot(a, b):
    return jnp.dot(a, b, preferred_element_type=jnp.float32,
                   precision=lax.Precision.HIGHEST)


def _tc_feat_body(x_ref, wg_ref, al_ref, ar_ref, feat_ref, el_ref, er_ref):
    f = _dot(x_ref[...], wg_ref[...])
    feat_ref[...] = f
    el_ref[...] = _dot(f, al_ref[...])
    er_ref[...] = _dot(f, ar_ref[...])


def _tc_feat(x, wg, al, ar):
    return pl.pallas_call(
        _tc_feat_body,
        out_shape=(
            jax.ShapeDtypeStruct((N, D_H), jnp.float32),
            jax.ShapeDtypeStruct((N, 1), jnp.float32),
            jax.ShapeDtypeStruct((N, 1), jnp.float32),
        ),
    )(x, wg, al, ar)


def _tc_h1_body(p_ref, s_ref, b_ref, o_ref):
    inv = 1.0 / (s_ref[0, :N] + s_ref[1, :N] + 1e-9)
    t = (p_ref[0, :N] + p_ref[1, :N]) * inv + b_ref[...]
    o_ref[...] = jnp.maximum(t, 0.0)


def _tc_h1(p, s, b_gat):
    return pl.pallas_call(
        _tc_h1_body,
        out_shape=jax.ShapeDtypeStruct((N, D_H), jnp.float32),
    )(p, s, b_gat)


def _tc_h2_body(q_ref, s_ref, w1_ref, b1_ref, o_ref):
    inv = 1.0 / (s_ref[0, :N] + s_ref[1, :N] + 1e-9)
    t2 = (q_ref[0, :N] + q_ref[1, :N]) * inv
    o_ref[...] = _dot(t2, w1_ref[...]) + b1_ref[...]


def _tc_h2(q, s, w1, b1):
    return pl.pallas_call(
        _tc_h2_body,
        out_shape=jax.ShapeDtypeStruct((N, D_H), jnp.float32),
    )(q, s, w1, b1)


def _tc_out_body(r_ref, s_ref, w2_ref, b2_ref, o_ref):
    inv = 1.0 / (s_ref[0, :N] + s_ref[1, :N] + 1e-9)
    o_ref[...] = _dot((r_ref[0, :N] + r_ref[1, :N]) * inv, w2_ref[...]) + b2_ref[...]


def _tc_out(r, s, w2, b2):
    return pl.pallas_call(
        _tc_out_body,
        out_shape=jax.ShapeDtypeStruct((N, D_OUT), jnp.float32),
    )(r, s, w2, b2)


def kernel(inputs, edge_index, W_gat, attn_l, attn_r, b_gat, W1, b1, W2, b2):
    pad = EPAD - E
    src3 = jnp.pad(edge_index[0], (0, pad)).reshape(NT, NCH, CH)
    dst3 = jnp.pad(edge_index[1], (0, pad)).reshape(NT, NCH, CH)
    zeros2 = jnp.zeros((NP, D_H), jnp.float32)
    zeros1 = jnp.zeros((NP,), jnp.float32)

    feat, el2, er2 = _tc_feat(inputs, W_gat,
                              attn_l.reshape(D_H, 1), attn_r.reshape(D_H, 1))
    el = el2.reshape(N)
    er = er2.reshape(N)

    ex3, s = _sc_attn(el, er, src3, dst3, zeros1)
    s3 = s.reshape(NC, NP, 1)

    p = _sc_prop(feat, src3, dst3, ex3, zeros2, D_H)
    h1 = _tc_h1(p, s3, b_gat.reshape(1, D_H))

    q = _sc_prop(h1, src3, dst3, ex3, zeros2, D_H)
    h2 = _tc_h2(q, s3, W1, b1.reshape(1, D_H))

    r = _sc_prop(h2, src3, dst3, ex3, zeros2, D_H)
    logits = _tc_out(r, s3, W2, b2.reshape(1, D_OUT))
    return logits
